# baseline (speedup 1.0000x reference)
import functools

import jax
import jax.numpy as jnp
from jax import lax
from jax.experimental import pallas as pl
from jax.experimental.pallas import tpu as pltpu
from jax.experimental.pallas import tpu_sc as plsc

_NC = 2
_NS = 16
_NW = _NC * _NS


@functools.lru_cache(maxsize=None)
def _build_gather(n_idx: int, d: int):
    b_per_w = n_idx // _NW
    chunk = 640
    while b_per_w % chunk:
        chunk //= 2
    n_chunks = b_per_w // chunk

    mesh = plsc.VectorSubcoreMesh(core_axis_name="c", subcore_axis_name="s")

    @functools.partial(
        pl.kernel,
        mesh=mesh,
        compiler_params=pltpu.CompilerParams(use_tc_tiling_on_sc=False),
        out_type=jax.ShapeDtypeStruct((n_idx, d), jnp.float32),
        scratch_types=[
            pltpu.VMEM((b_per_w,), jnp.int32),
            pltpu.VMEM((chunk, d), jnp.float32),
            pltpu.VMEM((chunk, d), jnp.float32),
            pltpu.SemaphoreType.DMA,
            pltpu.SemaphoreType.DMA,
            pltpu.SemaphoreType.DMA,
            pltpu.SemaphoreType.DMA,
        ],
    )
    def gather_kernel(table_hbm, idx_hbm, out_hbm, idx_v, buf0, buf1,
                      g0, g1, o0, o1):
        wid = lax.axis_index("s") * _NC + lax.axis_index("c")
        base = wid * b_per_w
        pltpu.sync_copy(idx_hbm.at[pl.ds(base, b_per_w)], idx_v)

        bufs = (buf0, buf1)
        gsems = (g0, g1)
        osems = (o0, o1)
        pending_out = [None, None]

        def start_gather(i):
            return pltpu.async_copy(
                table_hbm.at[idx_v.at[pl.ds(i * chunk, chunk)]],
                bufs[i % 2], gsems[i % 2])

        cur_gather = start_gather(0)
        for i in range(n_chunks):
            cur = i % 2
            nxt = (i + 1) % 2
            next_gather = None
            if i + 1 < n_chunks:
                if pending_out[nxt] is not None:
                    pending_out[nxt].wait()
                    pending_out[nxt] = None
                next_gather = start_gather(i + 1)
            cur_gather.wait()
            pending_out[cur] = pltpu.async_copy(
                bufs[cur], out_hbm.at[pl.ds(base + i * chunk, chunk)],
                osems[cur])
            cur_gather = next_gather
        for p in pending_out:
            if p is not None:
                p.wait()

    return gather_kernel


def kernel(x, table):
    b, l = x.shape
    _, d = table.shape
    flat = x.reshape(b * l)
    out = _build_gather(b * l, d)(table, flat)
    return out.reshape(b, l, d)
